# baseline (speedup 1.0000x reference)
import functools

import jax
import jax.numpy as jnp
from jax import lax
from jax.experimental import pallas as pl
from jax.experimental.pallas import tpu as pltpu
from jax.experimental.pallas import tpu_sc as plsc

B = 16384
HIST = 50
D = 16

NC = 2
NS = 16
NW = NC * NS
BPW = B // NW
CB = 64
NCHUNK = BPW // CB
ROWS = CB * HIST
NG = HIST // 2


def _sc_partials_body(xt_hbm, w_hbm, tab_hbm, out_hbm, w_v, idx_v, rows_v,
                      o_v, sem):
    cid = lax.axis_index("c")
    sid = lax.axis_index("s")
    wid = sid * NC + cid

    pltpu.sync_copy(w_hbm, w_v)

    def chunk_body(chunk, _):
        base = wid * BPW + chunk * CB
        pltpu.sync_copy(xt_hbm.at[:, pl.ds(base, CB)], idx_v)

        def fire(h, _):
            pltpu.async_copy(tab_hbm.at[idx_v.at[h]],
                             rows_v.at[pl.ds(h * CB, CB)], sem)
            return ()

        lax.fori_loop(0, HIST, fire, (), unroll=False)
        pltpu.make_async_copy(tab_hbm.at[pl.ds(0, ROWS)], rows_v, sem).wait()

        def b_body(bb, _):
            acc0 = jnp.zeros((D,), jnp.float32)
            acc1 = jnp.zeros((D,), jnp.float32)
            acc2 = jnp.zeros((D,), jnp.float32)
            for h in range(HIST):
                r = rows_v[h * CB + bb]
                acc0 = acc0 + r * w_v[h]
                acc1 = acc1 + r * w_v[HIST + h]
                acc2 = acc2 + r * w_v[2 * HIST + h]
            o_v[bb, pl.ds(0, D)] = acc0
            o_v[bb, pl.ds(D, D)] = acc1
            o_v[bb, pl.ds(2 * D, D)] = acc2
            return ()

        lax.fori_loop(0, CB, b_body, (), unroll=False)
        pltpu.sync_copy(o_v, out_hbm.at[pl.ds(base, CB)])
        return ()

    lax.fori_loop(0, NCHUNK, chunk_body, (), unroll=False)


TBC = 8192


def _tc_detile_body(i_ref, o_ref):
    t = i_ref[...]
    o_ref[...] = pltpu.einshape("d(rk)->r(kd)", t, k=8)


def _tc_finish_body(p_ref, b_ref, o_ref):
    p = p_ref[...]
    s0 = jnp.sum(p[:, 0:D], axis=1, keepdims=True)
    s1 = jnp.sum(p[:, D:2 * D], axis=1, keepdims=True)
    s2 = jnp.sum(p[:, 2 * D:3 * D], axis=1, keepdims=True)
    z = jnp.concatenate([s0, s1, s2], axis=1) + b_ref[...]
    m = jnp.max(z, axis=1, keepdims=True)
    e = jnp.exp(z - m)
    lse = jnp.log(jnp.sum(e, axis=1, keepdims=True))
    o_ref[...] = z - m - lse


@jax.jit
def kernel(x, table, W, b):
    xt = x.astype(jnp.int32).T
    w_r = W.astype(jnp.float32).reshape(3 * HIST, D)

    EMB = table.shape[0]
    n_tb = (EMB + TBC - 1) // TBC
    t128 = pl.pallas_call(
        _tc_detile_body,
        grid=(n_tb,),
        in_specs=[pl.BlockSpec((D, TBC), lambda i: (0, i))],
        out_specs=pl.BlockSpec((TBC // 8, 128), lambda i: (i, 0)),
        out_shape=jax.ShapeDtypeStruct((EMB // 8, 128), jnp.float32),
    )(table.T)
    tab_lin = t128.reshape(EMB, D)

    mesh = plsc.VectorSubcoreMesh(core_axis_name="c", subcore_axis_name="s")
    sc_fn = functools.partial(
        pl.kernel,
        out_type=jax.ShapeDtypeStruct((B, 3 * D), jnp.float32),
        mesh=mesh,
        scratch_types=[
            pltpu.VMEM((3 * HIST, D), jnp.float32),
            pltpu.VMEM((HIST, CB), jnp.int32),
            pltpu.VMEM((ROWS, D), jnp.float32),
            pltpu.VMEM((CB, 3 * D), jnp.float32),
            pltpu.SemaphoreType.DMA,
        ],
        compiler_params=pltpu.CompilerParams(use_tc_tiling_on_sc=False),
    )(_sc_partials_body)
    partials = sc_fn(xt, w_r, tab_lin)

    out = pl.pallas_call(
        _tc_finish_body,
        out_shape=jax.ShapeDtypeStruct((B, 3), jnp.float32),
    )(partials, b.reshape(1, 3))
    return out


# speedup vs baseline: 1.9451x; 1.9451x over previous
import functools

import jax
import jax.numpy as jnp
from jax import lax
from jax.experimental import pallas as pl
from jax.experimental.pallas import tpu as pltpu
from jax.experimental.pallas import tpu_sc as plsc

B = 16384
HIST = 50
D = 16

NC = 2
NS = 16
NW = NC * NS
BPW = B // NW
CB = 64
NCHUNK = BPW // CB
ROWS = CB * HIST
G = 4


def _sc_partials_body(xt_hbm, w_hbm, tab_hbm, out_hbm, w_v, idx_v, rows_v,
                      o_v, sems):
    cid = lax.axis_index("c")
    sid = lax.axis_index("s")
    wid = sid * NC + cid

    pltpu.sync_copy(w_hbm, w_v)

    def fire(chunk, buf):
        base = wid * BPW + chunk * CB
        pltpu.sync_copy(xt_hbm.at[:, pl.ds(base, CB)], idx_v.at[buf])

        def fire_one(h, _):
            pltpu.async_copy(tab_hbm.at[idx_v.at[buf, h]],
                             rows_v.at[buf, pl.ds(h * CB, CB)], sems.at[buf])
            return ()

        lax.fori_loop(0, HIST, fire_one, (), unroll=False)

    def drain(buf):
        pltpu.make_async_copy(tab_hbm.at[pl.ds(0, ROWS)], rows_v.at[buf],
                              sems.at[buf]).wait()

    def compute(chunk, buf):
        def b_body(bg, _):
            b0 = bg * G
            accs = [[jnp.zeros((D,), jnp.float32) for _ in range(G)]
                    for _ in range(3)]
            for h in range(HIST):
                ws = [w_v[c * HIST + h] for c in range(3)]
                for g in range(G):
                    r = rows_v[buf, h * CB + b0 + g]
                    for c in range(3):
                        accs[c][g] = accs[c][g] + r * ws[c]
            for g in range(G):
                for c in range(3):
                    o_v[b0 + g, pl.ds(c * D, D)] = accs[c][g]
            return ()

        lax.fori_loop(0, CB // G, b_body, (), unroll=False)
        base = wid * BPW + chunk * CB
        pltpu.sync_copy(o_v, out_hbm.at[pl.ds(base, CB)])

    fire(0, 0)

    def pipe(j, _):
        c0 = 2 * j
        fire(c0 + 1, 1)
        drain(0)
        compute(c0, 0)

        @pl.when(c0 + 2 < NCHUNK)
        def _():
            fire(c0 + 2, 0)

        drain(1)
        compute(c0 + 1, 1)
        return ()

    lax.fori_loop(0, NCHUNK // 2, pipe, (), unroll=False)


def _tc_finish_body(p_ref, b_ref, o_ref):
    p = p_ref[...]
    s0 = jnp.sum(p[:, 0:D], axis=1, keepdims=True)
    s1 = jnp.sum(p[:, D:2 * D], axis=1, keepdims=True)
    s2 = jnp.sum(p[:, 2 * D:3 * D], axis=1, keepdims=True)
    z = jnp.concatenate([s0, s1, s2], axis=1) + b_ref[...]
    m = jnp.max(z, axis=1, keepdims=True)
    e = jnp.exp(z - m)
    lse = jnp.log(jnp.sum(e, axis=1, keepdims=True))
    o_ref[...] = z - m - lse


@jax.jit
def kernel(x, table, W, b):
    xt = x.astype(jnp.int32).T
    w_r = W.astype(jnp.float32).reshape(3 * HIST, D)

    mesh = plsc.VectorSubcoreMesh(core_axis_name="c", subcore_axis_name="s")
    sc_fn = functools.partial(
        pl.kernel,
        out_type=jax.ShapeDtypeStruct((B, 3 * D), jnp.float32),
        mesh=mesh,
        scratch_types=[
            pltpu.VMEM((3 * HIST, D), jnp.float32),
            pltpu.VMEM((2, HIST, CB), jnp.int32),
            pltpu.VMEM((2, ROWS, D), jnp.float32),
            pltpu.VMEM((CB, 3 * D), jnp.float32),
            pltpu.SemaphoreType.DMA((2,)),
        ],
        compiler_params=pltpu.CompilerParams(use_tc_tiling_on_sc=False),
    )(_sc_partials_body)
    partials = sc_fn(xt, w_r, table)

    out = pl.pallas_call(
        _tc_finish_body,
        out_shape=jax.ShapeDtypeStruct((B, 3), jnp.float32),
    )(partials, b.reshape(1, 3))
    return out


# speedup vs baseline: 1.9561x; 1.0056x over previous
import functools

import jax
import jax.numpy as jnp
from jax import lax
from jax.experimental import pallas as pl
from jax.experimental.pallas import tpu as pltpu
from jax.experimental.pallas import tpu_sc as plsc

B = 16384
HIST = 50
D = 16

NC = 2
NS = 16
NW = NC * NS
BPW = B // NW
CB = 64
NCHUNK = BPW // CB
ROWS = CB * HIST
G = 4


def _sc_partials_body(xt_hbm, w_hbm, tab_hbm, out_hbm, w_v, idx_v, rows_v,
                      o_v, sems):
    cid = lax.axis_index("c")
    sid = lax.axis_index("s")
    wid = sid * NC + cid

    pltpu.sync_copy(w_hbm, w_v)

    def fire(chunk, buf):
        base = wid * BPW + chunk * CB
        pltpu.sync_copy(xt_hbm.at[:, pl.ds(base, CB)], idx_v.at[buf])

        def fire_one(h, _):
            pltpu.async_copy(tab_hbm.at[idx_v.at[buf, h]],
                             rows_v.at[buf, pl.ds(h * CB, CB)], sems.at[buf])
            return ()

        lax.fori_loop(0, HIST, fire_one, (), unroll=False)

    def drain(buf):
        pltpu.make_async_copy(tab_hbm.at[pl.ds(0, ROWS)], rows_v.at[buf],
                              sems.at[buf]).wait()

    def compute(chunk, buf):
        def b_body(bg, _):
            b0 = bg * G
            accs = [[jnp.zeros((D,), jnp.float32) for _ in range(G)]
                    for _ in range(3)]
            for h in range(HIST):
                ws = [w_v[c * HIST + h] for c in range(3)]
                for g in range(G):
                    r = rows_v[buf, h * CB + b0 + g]
                    for c in range(3):
                        accs[c][g] = accs[c][g] + r * ws[c]
            for g in range(G):
                for c in range(3):
                    o_v[b0 + g, pl.ds(c * D, D)] = accs[c][g]
            return ()

        lax.fori_loop(0, CB // G, b_body, (), unroll=False)
        base = wid * BPW + chunk * CB
        pltpu.sync_copy(o_v, out_hbm.at[pl.ds(base, CB)])

    fire(0, 0)

    def pipe(j, _):
        c0 = 2 * j
        fire(c0 + 1, 1)
        drain(0)
        compute(c0, 0)

        @pl.when(c0 + 2 < NCHUNK)
        def _():
            fire(c0 + 2, 0)

        drain(1)
        compute(c0 + 1, 1)
        return ()

    lax.fori_loop(0, NCHUNK // 2, pipe, (), unroll=False)


def _tc_finish_body(p_ref, b_ref, o_ref):
    p = p_ref[...]
    s0 = jnp.sum(p[:, 0:D], axis=1, keepdims=True)
    s1 = jnp.sum(p[:, D:2 * D], axis=1, keepdims=True)
    s2 = jnp.sum(p[:, 2 * D:3 * D], axis=1, keepdims=True)
    z = jnp.concatenate([s0, s1, s2], axis=1) + b_ref[...]
    m = jnp.max(z, axis=1, keepdims=True)
    e = jnp.exp(z - m)
    lse = jnp.log(jnp.sum(e, axis=1, keepdims=True))
    o_ref[...] = z - m - lse


@jax.jit
def kernel(x, table, W, b):
    xt = x.astype(jnp.int32).T
    w_r = W.astype(jnp.float32).reshape(3 * HIST, D)

    mesh = plsc.VectorSubcoreMesh(core_axis_name="c", subcore_axis_name="s")
    sc_fn = functools.partial(
        pl.kernel,
        out_type=jax.ShapeDtypeStruct((B, 3 * D), jnp.float32),
        mesh=mesh,
        scratch_types=[
            pltpu.VMEM((3 * HIST, D), jnp.float32),
            pltpu.VMEM((2, HIST, CB), jnp.int32),
            pltpu.VMEM((2, ROWS, D), jnp.float32),
            pltpu.VMEM((CB, 3 * D), jnp.float32),
            pltpu.SemaphoreType.DMA((2,)),
        ],
        compiler_params=pltpu.CompilerParams(use_tc_tiling_on_sc=False),
    )(_sc_partials_body)
    partials = sc_fn(xt, w_r, table)

    FB = 2048
    out = pl.pallas_call(
        _tc_finish_body,
        grid=(B // FB,),
        in_specs=[pl.BlockSpec((FB, 3 * D), lambda i: (i, 0)),
                  pl.BlockSpec((1, 3), lambda i: (0, 0))],
        out_specs=pl.BlockSpec((FB, 3), lambda i: (i, 0)),
        out_shape=jax.ShapeDtypeStruct((B, 3), jnp.float32),
    )(partials, b.reshape(1, 3))
    return out
